# baseline (speedup 1.0000x reference)
import jax
import jax.numpy as jnp
from jax.experimental import pallas as pl
from jax.experimental.pallas import tpu as pltpu
from jax.experimental.pallas import tpu_sc as plsc

_DIM = 768
_HIDDEN = 1024
_E = 8
_TOKENS = 8192
_Z_COEF = 0.001
_LOAD_COEF = 0.001

_TB = 256
_NBLK = _TOKENS // _TB
_TBR = 1024
_NBR = _TOKENS // _TBR
_CAP_BLOCKS = _NBLK + _E - 1
_CAP = _CAP_BLOCKS * _TB
_SCW = 128


def _route_body(x_ref, rw_ref, pos_ref, bg_ref, aux_ref, xb16_ref,
                e_scr, rank_scr, cnt_scr, lse_scr, load_scr):
    b = pl.program_id(0)

    @pl.when(b == 0)
    def _init():
        cnt_scr[...] = jnp.zeros_like(cnt_scr)
        lse_scr[...] = jnp.zeros_like(lse_scr)
        load_scr[...] = jnp.zeros_like(load_scr)

    @pl.when(b < _NBR)
    def _block():
        xb = x_ref[...]
        xb16_ref[...] = xb.astype(jnp.bfloat16)
        rw = rw_ref[...]
        logits = jax.lax.dot_general(
            rw, xb, (((1,), (1,)), ((), ())))
        m = jnp.max(logits, axis=0, keepdims=True)
        iota_e = jax.lax.broadcasted_iota(jnp.int32, (_E, _TBR), 0)
        e = jnp.min(jnp.where(logits == m, iota_e, _E), axis=0,
                    keepdims=True)
        ex = jnp.exp(logits - m)
        s = jnp.sum(ex, axis=0, keepdims=True)
        lse_scr[...] += m + jnp.log(s)
        load_scr[...] += ex / s
        oh = (iota_e == e).astype(jnp.float32)
        jr = jax.lax.broadcasted_iota(jnp.int32, (_TBR, _TBR), 0)
        ir = jax.lax.broadcasted_iota(jnp.int32, (_TBR, _TBR), 1)
        tri = (jr < ir).astype(jnp.float32)
        rank_ex = jax.lax.dot_general(
            oh, tri, (((1,), (0,)), ((), ())),
            preferred_element_type=jnp.float32)
        cnt = cnt_scr[...]
        rank_g = jnp.sum(oh * (rank_ex + cnt), axis=0,
                         keepdims=True)
        cnt_scr[...] = cnt + jnp.sum(oh, axis=1, keepdims=True)
        e_scr[pl.ds(b, 1), :] = e
        rank_scr[pl.ds(b, 1), :] = rank_g

    @pl.when(b == _NBR)
    def _fin():
        xb16_ref[...] = x_ref[...].astype(jnp.bfloat16)
        cnt = cnt_scr[...]
        pc = jnp.ceil(cnt / _TB) * _TB
        ar = jax.lax.broadcasted_iota(jnp.int32, (_E, _E), 0)
        ac = jax.lax.broadcasted_iota(jnp.int32, (_E, _E), 1)
        low = (ac < ar).astype(jnp.float32)
        off = jax.lax.dot_general(
            low, pc, (((1,), (0,)), ((), ())),
            preferred_element_type=jnp.float32)

        bl = (jax.lax.broadcasted_iota(jnp.int32, (1, 128), 1)
              .astype(jnp.float32) * _TB)
        ge = jnp.sum(((off + pc) <= bl).astype(jnp.int32), axis=0,
                     keepdims=True)
        bg_ref[0:1, :] = jnp.minimum(ge, _E - 1)

        lse_sum = jnp.sum(lse_scr[...], axis=1, keepdims=True)
        mean_lse = lse_sum / _TOKENS
        z_loss = mean_lse * mean_lse
        load = jnp.sum(load_scr[...], axis=1, keepdims=True)
        mean_l = jnp.sum(load, axis=0, keepdims=True) / _E
        var = jnp.sum((load - mean_l) ** 2, axis=0,
                      keepdims=True) / (_E - 1)
        cv2 = var / (mean_l * mean_l)
        aux_ref[...] = _Z_COEF * z_loss + _LOAD_COEF * cv2

        ev = e_scr[...]
        p = rank_scr[...]
        for k in range(_E):
            p += jnp.where(ev == k, off[k:k + 1, 0:1], 0.0)
        pos_ref[...] = p.astype(jnp.int32)


def _route(x, router_w):
    return pl.pallas_call(
        _route_body,
        grid=(_NBR + 1,),
        in_specs=[
            pl.BlockSpec((_TBR, _DIM), lambda i: (jnp.minimum(i, _NBR - 1), 0)),
            pl.BlockSpec((_E, _DIM), lambda i: (0, 0)),
        ],
        out_specs=[
            pl.BlockSpec((_NBR, _TBR), lambda i: (0, 0)),
            pl.BlockSpec((8, 128), lambda i: (0, 0)),
            pl.BlockSpec((1, 1), lambda i: (0, 0)),
            pl.BlockSpec((_TBR, _DIM), lambda i: (jnp.minimum(i, _NBR - 1), 0)),
        ],
        out_shape=[
            jax.ShapeDtypeStruct((_NBR, _TBR), jnp.int32),
            jax.ShapeDtypeStruct((8, 128), jnp.int32),
            jax.ShapeDtypeStruct((1, 1), jnp.float32),
            jax.ShapeDtypeStruct((_TOKENS, _DIM), jnp.bfloat16),
        ],
        scratch_shapes=[
            pltpu.VMEM((_NBR, _TBR), jnp.int32),
            pltpu.VMEM((_NBR, _TBR), jnp.float32),
            pltpu.VMEM((_E, 1), jnp.float32),
            pltpu.VMEM((1, _TBR), jnp.float32),
            pltpu.VMEM((_E, _TBR), jnp.float32),
        ],
    )(x, router_w)


def _moe_body(bg_ref, xs_ref, w1_ref, w2_ref, w3_ref, ys_ref):
    del bg_ref
    xb = xs_ref[...].astype(jnp.float32)
    h1 = jax.lax.dot_general(
        xb, w1_ref[0], (((1,), (1,)), ((), ())))
    h2 = jax.lax.dot_general(
        xb, w2_ref[0], (((1,), (1,)), ((), ())))
    a = h1 * jax.lax.logistic(h1) * h2
    ys_ref[...] = jax.lax.dot_general(
        a, w3_ref[0], (((1,), (1,)), ((), ())))


def _moe(bg, xs, w1, w2, w3):
    grid_spec = pltpu.PrefetchScalarGridSpec(
        num_scalar_prefetch=1,
        grid=(_CAP_BLOCKS,),
        in_specs=[
            pl.BlockSpec((_TB, _DIM), lambda i, bg: (i, 0)),
            pl.BlockSpec((1, _HIDDEN, _DIM), lambda i, bg: (bg[i], 0, 0)),
            pl.BlockSpec((1, _HIDDEN, _DIM), lambda i, bg: (bg[i], 0, 0)),
            pl.BlockSpec((1, _DIM, _HIDDEN), lambda i, bg: (bg[i], 0, 0)),
        ],
        out_specs=pl.BlockSpec((_TB, _DIM), lambda i, bg: (i, 0)),
    )
    return pl.pallas_call(
        _moe_body,
        grid_spec=grid_spec,
        out_shape=jax.ShapeDtypeStruct((_CAP, _DIM), jnp.float32),
        compiler_params=pltpu.CompilerParams(
            dimension_semantics=("parallel",)),
    )(bg, xs, w1, w2, w3)


_NC = 2
_NS = 16
_NW = _NC * _NS
_PER_W = _TOKENS // _NW
_CHUNK = 128


def _vector_mesh():
    return plsc.VectorSubcoreMesh(
        core_axis_name="core", subcore_axis_name="subcore")


def _sc_scatter(x, pos1d):
    @pl.kernel(out_type=jax.ShapeDtypeStruct((_CAP, _DIM // 2), jnp.int32),
               mesh=_vector_mesh(),
               scratch_types=[
                   pltpu.VMEM((_CHUNK,), jnp.int32),
                   pltpu.VMEM((_CHUNK, _DIM // 2), jnp.int32),
                   pltpu.SemaphoreType.DMA,
               ])
    def kern(x_hbm, i_hbm, xs_hbm, idx_v, rows_v, sem):
        wid = (jax.lax.axis_index("core") * _NS
               + jax.lax.axis_index("subcore"))
        base = wid * _PER_W

        @pl.loop(0, _PER_W // _CHUNK)
        def _(j):
            b = base + j * _CHUNK
            pltpu.sync_copy(i_hbm.at[pl.ds(b, _CHUNK)], idx_v)
            pltpu.sync_copy(x_hbm.at[pl.ds(b, _CHUNK)], rows_v)
            pltpu.async_copy(rows_v, xs_hbm.at[idx_v], sem).wait()

    return kern(x, pos1d)


def _sc_gather(ys, pos1d):
    @pl.kernel(out_type=jax.ShapeDtypeStruct((_TOKENS, _DIM), jnp.float32),
               mesh=_vector_mesh(),
               scratch_types=[
                   pltpu.VMEM((_CHUNK,), jnp.int32),
                   pltpu.VMEM((_CHUNK, _DIM), jnp.float32),
                   pltpu.SemaphoreType.DMA,
               ])
    def kern(y_hbm, i_hbm, o_hbm, idx_v, rows_v, sem):
        wid = (jax.lax.axis_index("core") * _NS
               + jax.lax.axis_index("subcore"))
        base = wid * _PER_W

        @pl.loop(0, _PER_W // _CHUNK)
        def _(j):
            b = base + j * _CHUNK
            pltpu.sync_copy(i_hbm.at[pl.ds(b, _CHUNK)], idx_v)
            pltpu.async_copy(y_hbm.at[idx_v], rows_v, sem).wait()
            pltpu.sync_copy(rows_v, o_hbm.at[pl.ds(b, _CHUNK)])

    return kern(ys, pos1d)


def kernel(x, router_w, w1, w2, w3):
    pos, meta, aux, x16 = _route(x, router_w)
    pos1d = pos.reshape(_TOKENS)
    bg = meta[0, :_CAP_BLOCKS]
    x16i = jax.lax.bitcast_convert_type(
        x16.reshape(_TOKENS, _DIM // 2, 2), jnp.int32)
    xsi = _sc_scatter(x16i, pos1d)
    xs = jax.lax.bitcast_convert_type(xsi, jnp.bfloat16).reshape(_CAP, _DIM)
    ys = _moe(bg, xs, w1, w2, w3)
    out = _sc_gather(ys, pos1d)
    return out, aux[0, 0]


# speedup vs baseline: 2.7713x; 2.7713x over previous
import jax
import jax.numpy as jnp
from jax.experimental import pallas as pl
from jax.experimental.pallas import tpu as pltpu
from jax.experimental.pallas import tpu_sc as plsc

_DIM = 768
_HIDDEN = 1024
_E = 8
_TOKENS = 8192
_Z_COEF = 0.001
_LOAD_COEF = 0.001

_TB = 256
_NBLK = _TOKENS // _TB
_TBR = 1024
_NBR = _TOKENS // _TBR
_CAP_BLOCKS = _NBLK + _E - 1
_CAP = _CAP_BLOCKS * _TB
_SCW = 128


def _route_body(x_ref, rw_ref, pos_ref, bg_ref, aux_ref,
                e_scr, rank_scr, cnt_scr, lse_scr, load_scr):
    b = pl.program_id(0)

    @pl.when(b == 0)
    def _init():
        cnt_scr[...] = jnp.zeros_like(cnt_scr)
        lse_scr[...] = jnp.zeros_like(lse_scr)
        load_scr[...] = jnp.zeros_like(load_scr)

    @pl.when(b < _NBR)
    def _block():
        xb = x_ref[...]
        rw = rw_ref[...]
        logits = jax.lax.dot_general(
            rw, xb, (((1,), (1,)), ((), ())))
        m = jnp.max(logits, axis=0, keepdims=True)
        iota_e = jax.lax.broadcasted_iota(jnp.int32, (_E, _TBR), 0)
        e = jnp.min(jnp.where(logits == m, iota_e, _E), axis=0,
                    keepdims=True)
        ex = jnp.exp(logits - m)
        s = jnp.sum(ex, axis=0, keepdims=True)
        lse_scr[...] += m + jnp.log(s)
        load_scr[...] += ex / s
        oh = (iota_e == e).astype(jnp.float32)
        jr = jax.lax.broadcasted_iota(jnp.int32, (_TBR, _TBR), 0)
        ir = jax.lax.broadcasted_iota(jnp.int32, (_TBR, _TBR), 1)
        tri = (jr < ir).astype(jnp.float32)
        rank_ex = jax.lax.dot_general(
            oh, tri, (((1,), (0,)), ((), ())),
            preferred_element_type=jnp.float32)
        cnt = cnt_scr[...]
        rank_g = jnp.sum(oh * (rank_ex + cnt), axis=0,
                         keepdims=True)
        cnt_scr[...] = cnt + jnp.sum(oh, axis=1, keepdims=True)
        e_scr[pl.ds(b, 1), :] = e
        rank_scr[pl.ds(b, 1), :] = rank_g

    @pl.when(b == _NBR)
    def _fin():
        cnt = cnt_scr[...]
        pc = jnp.ceil(cnt / _TB) * _TB
        ar = jax.lax.broadcasted_iota(jnp.int32, (_E, _E), 0)
        ac = jax.lax.broadcasted_iota(jnp.int32, (_E, _E), 1)
        low = (ac < ar).astype(jnp.float32)
        off = jax.lax.dot_general(
            low, pc, (((1,), (0,)), ((), ())),
            preferred_element_type=jnp.float32)

        bl = (jax.lax.broadcasted_iota(jnp.int32, (1, 128), 1)
              .astype(jnp.float32) * _TB)
        ge = jnp.sum(((off + pc) <= bl).astype(jnp.int32), axis=0,
                     keepdims=True)
        bg_ref[0:1, :] = jnp.minimum(ge, _E - 1)

        lse_sum = jnp.sum(lse_scr[...], axis=1, keepdims=True)
        mean_lse = lse_sum / _TOKENS
        z_loss = mean_lse * mean_lse
        load = jnp.sum(load_scr[...], axis=1, keepdims=True)
        mean_l = jnp.sum(load, axis=0, keepdims=True) / _E
        var = jnp.sum((load - mean_l) ** 2, axis=0,
                      keepdims=True) / (_E - 1)
        cv2 = var / (mean_l * mean_l)
        aux_ref[...] = _Z_COEF * z_loss + _LOAD_COEF * cv2

        ev = e_scr[...]
        p = rank_scr[...]
        for k in range(_E):
            p += jnp.where(ev == k, off[k:k + 1, 0:1], 0.0)
        pos_ref[...] = p.astype(jnp.int32)


def _route(x, router_w):
    return pl.pallas_call(
        _route_body,
        grid=(_NBR + 1,),
        in_specs=[
            pl.BlockSpec((_TBR, _DIM), lambda i: (jnp.minimum(i, _NBR - 1), 0)),
            pl.BlockSpec((_E, _DIM), lambda i: (0, 0)),
        ],
        out_specs=[
            pl.BlockSpec((_NBR, _TBR), lambda i: (0, 0)),
            pl.BlockSpec((8, 128), lambda i: (0, 0)),
            pl.BlockSpec((1, 1), lambda i: (0, 0)),
        ],
        out_shape=[
            jax.ShapeDtypeStruct((_NBR, _TBR), jnp.int32),
            jax.ShapeDtypeStruct((8, 128), jnp.int32),
            jax.ShapeDtypeStruct((1, 1), jnp.float32),
        ],
        scratch_shapes=[
            pltpu.VMEM((_NBR, _TBR), jnp.int32),
            pltpu.VMEM((_NBR, _TBR), jnp.float32),
            pltpu.VMEM((_E, 1), jnp.float32),
            pltpu.VMEM((1, _TBR), jnp.float32),
            pltpu.VMEM((_E, _TBR), jnp.float32),
        ],
    )(x, router_w)


def _moe_body(bg_ref, xs_ref, w1_ref, w2_ref, w3_ref, ys_ref):
    del bg_ref
    xb = xs_ref[...]
    h1 = jax.lax.dot_general(
        xb, w1_ref[0], (((1,), (1,)), ((), ())))
    h2 = jax.lax.dot_general(
        xb, w2_ref[0], (((1,), (1,)), ((), ())))
    a = h1 * jax.lax.logistic(h1) * h2
    ys_ref[...] = jax.lax.dot_general(
        a, w3_ref[0], (((1,), (1,)), ((), ())))


def _moe(bg, xs, w1, w2, w3):
    grid_spec = pltpu.PrefetchScalarGridSpec(
        num_scalar_prefetch=1,
        grid=(_CAP_BLOCKS,),
        in_specs=[
            pl.BlockSpec((_TB, _DIM), lambda i, bg: (i, 0)),
            pl.BlockSpec((1, _HIDDEN, _DIM), lambda i, bg: (bg[i], 0, 0)),
            pl.BlockSpec((1, _HIDDEN, _DIM), lambda i, bg: (bg[i], 0, 0)),
            pl.BlockSpec((1, _DIM, _HIDDEN), lambda i, bg: (bg[i], 0, 0)),
        ],
        out_specs=pl.BlockSpec((_TB, _DIM), lambda i, bg: (i, 0)),
    )
    return pl.pallas_call(
        _moe_body,
        grid_spec=grid_spec,
        out_shape=jax.ShapeDtypeStruct((_CAP, _DIM), jnp.float32),
        compiler_params=pltpu.CompilerParams(
            dimension_semantics=("parallel",)),
    )(bg, xs, w1, w2, w3)


_NC = 2
_NS = 16
_NW = _NC * _NS
_PER_W = _TOKENS // _NW
_CHUNK = 128


def _vector_mesh():
    return plsc.VectorSubcoreMesh(
        core_axis_name="core", subcore_axis_name="subcore")


def _sc_scatter(x, pos1d):
    @pl.kernel(out_type=jax.ShapeDtypeStruct((_CAP, _DIM), jnp.float32),
               mesh=_vector_mesh(),
               scratch_types=[
                   pltpu.VMEM((_CHUNK,), jnp.int32),
                   pltpu.VMEM((_CHUNK, _DIM), jnp.float32),
                   pltpu.SemaphoreType.DMA,
               ])
    def kern(x_hbm, i_hbm, xs_hbm, idx_v, rows_v, sem):
        wid = (jax.lax.axis_index("core") * _NS
               + jax.lax.axis_index("subcore"))
        base = wid * _PER_W

        @pl.loop(0, _PER_W // _CHUNK)
        def _(j):
            b = base + j * _CHUNK
            pltpu.sync_copy(i_hbm.at[pl.ds(b, _CHUNK)], idx_v)
            pltpu.sync_copy(x_hbm.at[pl.ds(b, _CHUNK)], rows_v)
            pltpu.async_copy(rows_v, xs_hbm.at[idx_v], sem).wait()

    return kern(x, pos1d)


def _sc_gather(ys, pos1d):
    @pl.kernel(out_type=jax.ShapeDtypeStruct((_TOKENS, _DIM), jnp.float32),
               mesh=_vector_mesh(),
               scratch_types=[
                   pltpu.VMEM((_CHUNK,), jnp.int32),
                   pltpu.VMEM((_CHUNK, _DIM), jnp.float32),
                   pltpu.SemaphoreType.DMA,
               ])
    def kern(y_hbm, i_hbm, o_hbm, idx_v, rows_v, sem):
        wid = (jax.lax.axis_index("core") * _NS
               + jax.lax.axis_index("subcore"))
        base = wid * _PER_W

        @pl.loop(0, _PER_W // _CHUNK)
        def _(j):
            b = base + j * _CHUNK
            pltpu.sync_copy(i_hbm.at[pl.ds(b, _CHUNK)], idx_v)
            pltpu.async_copy(y_hbm.at[idx_v], rows_v, sem).wait()
            pltpu.sync_copy(rows_v, o_hbm.at[pl.ds(b, _CHUNK)])

    return kern(ys, pos1d)


def kernel(x, router_w, w1, w2, w3):
    pos, meta, aux = _route(x, router_w)
    pos1d = pos.reshape(_TOKENS)
    bg = meta[0, :_CAP_BLOCKS]
    xs = _sc_scatter(x, pos1d)
    ys = _moe(bg, xs, w1, w2, w3)
    out = _sc_gather(ys, pos1d)
    return out, aux[0, 0]


# speedup vs baseline: 2.7723x; 1.0004x over previous
import jax
import jax.numpy as jnp
from jax.experimental import pallas as pl
from jax.experimental.pallas import tpu as pltpu
from jax.experimental.pallas import tpu_sc as plsc

_DIM = 768
_HIDDEN = 1024
_E = 8
_TOKENS = 8192
_Z_COEF = 0.001
_LOAD_COEF = 0.001

_TB = 256
_NBLK = _TOKENS // _TB
_TBR = 1024
_NBR = _TOKENS // _TBR
_CAP_BLOCKS = _NBLK + _E - 1
_CAP = _CAP_BLOCKS * _TB
_SCW = 128


def _route_body(x_ref, rw_ref, pos_ref, bg_ref, aux_ref,
                e_scr, rank_scr, cnt_scr, lse_scr, load_scr):
    b = pl.program_id(0)

    @pl.when(b == 0)
    def _init():
        cnt_scr[...] = jnp.zeros_like(cnt_scr)
        lse_scr[...] = jnp.zeros_like(lse_scr)
        load_scr[...] = jnp.zeros_like(load_scr)

    @pl.when(b < _NBR)
    def _block():
        xb = x_ref[...]
        rw = rw_ref[...]
        logits = jax.lax.dot_general(
            rw, xb, (((1,), (1,)), ((), ())))
        m = jnp.max(logits, axis=0, keepdims=True)
        iota_e = jax.lax.broadcasted_iota(jnp.int32, (_E, _TBR), 0)
        e = jnp.min(jnp.where(logits == m, iota_e, _E), axis=0,
                    keepdims=True)
        ex = jnp.exp(logits - m)
        s = jnp.sum(ex, axis=0, keepdims=True)
        lse_scr[...] += m + jnp.log(s)
        load_scr[...] += ex / s
        oh = (iota_e == e).astype(jnp.float32)
        jr = jax.lax.broadcasted_iota(jnp.int32, (_TBR, _TBR), 0)
        ir = jax.lax.broadcasted_iota(jnp.int32, (_TBR, _TBR), 1)
        tri = (jr < ir).astype(jnp.float32)
        rank_ex = jax.lax.dot_general(
            oh, tri, (((1,), (0,)), ((), ())),
            preferred_element_type=jnp.float32)
        cnt = cnt_scr[...]
        rank_g = jnp.sum(oh * (rank_ex + cnt), axis=0,
                         keepdims=True)
        cnt_scr[...] = cnt + jnp.sum(oh, axis=1, keepdims=True)
        e_scr[pl.ds(b, 1), :] = e
        rank_scr[pl.ds(b, 1), :] = rank_g

    @pl.when(b == _NBR)
    def _fin():
        cnt = cnt_scr[...]
        pc = jnp.ceil(cnt / _TB) * _TB
        ar = jax.lax.broadcasted_iota(jnp.int32, (_E, _E), 0)
        ac = jax.lax.broadcasted_iota(jnp.int32, (_E, _E), 1)
        low = (ac < ar).astype(jnp.float32)
        off = jax.lax.dot_general(
            low, pc, (((1,), (0,)), ((), ())),
            preferred_element_type=jnp.float32)

        bl = (jax.lax.broadcasted_iota(jnp.int32, (1, 128), 1)
              .astype(jnp.float32) * _TB)
        ge = jnp.sum(((off + pc) <= bl).astype(jnp.int32), axis=0,
                     keepdims=True)
        bg_ref[0:1, :] = jnp.minimum(ge, _E - 1)

        lse_sum = jnp.sum(lse_scr[...], axis=1, keepdims=True)
        mean_lse = lse_sum / _TOKENS
        z_loss = mean_lse * mean_lse
        load = jnp.sum(load_scr[...], axis=1, keepdims=True)
        mean_l = jnp.sum(load, axis=0, keepdims=True) / _E
        var = jnp.sum((load - mean_l) ** 2, axis=0,
                      keepdims=True) / (_E - 1)
        cv2 = var / (mean_l * mean_l)
        aux_ref[...] = _Z_COEF * z_loss + _LOAD_COEF * cv2

        ev = e_scr[...]
        p = rank_scr[...]
        for k in range(_E):
            p += jnp.where(ev == k, off[k:k + 1, 0:1], 0.0)
        pos_ref[...] = p.astype(jnp.int32)


def _route(x, router_w):
    return pl.pallas_call(
        _route_body,
        grid=(_NBR + 1,),
        in_specs=[
            pl.BlockSpec((_TBR, _DIM), lambda i: (jnp.minimum(i, _NBR - 1), 0)),
            pl.BlockSpec((_E, _DIM), lambda i: (0, 0)),
        ],
        out_specs=[
            pl.BlockSpec((_NBR, _TBR), lambda i: (0, 0)),
            pl.BlockSpec((8, 128), lambda i: (0, 0)),
            pl.BlockSpec((1, 1), lambda i: (0, 0)),
        ],
        out_shape=[
            jax.ShapeDtypeStruct((_NBR, _TBR), jnp.int32),
            jax.ShapeDtypeStruct((8, 128), jnp.int32),
            jax.ShapeDtypeStruct((1, 1), jnp.float32),
        ],
        scratch_shapes=[
            pltpu.VMEM((_NBR, _TBR), jnp.int32),
            pltpu.VMEM((_NBR, _TBR), jnp.float32),
            pltpu.VMEM((_E, 1), jnp.float32),
            pltpu.VMEM((1, _TBR), jnp.float32),
            pltpu.VMEM((_E, _TBR), jnp.float32),
        ],
    )(x, router_w)


def _moe_body(bg_ref, xs_ref, w1_ref, w2_ref, w3_ref, ys_ref):
    del bg_ref
    xb = xs_ref[...]
    h1 = jax.lax.dot_general(
        xb, w1_ref[0], (((1,), (1,)), ((), ())))
    h2 = jax.lax.dot_general(
        xb, w2_ref[0], (((1,), (1,)), ((), ())))
    a = h1 * jax.lax.logistic(h1) * h2
    ys_ref[...] = jax.lax.dot_general(
        a, w3_ref[0], (((1,), (1,)), ((), ())))


def _moe(bg, xs, w1, w2, w3):
    grid_spec = pltpu.PrefetchScalarGridSpec(
        num_scalar_prefetch=1,
        grid=(_CAP_BLOCKS,),
        in_specs=[
            pl.BlockSpec((_TB, _DIM), lambda i, bg: (i, 0)),
            pl.BlockSpec((1, _HIDDEN, _DIM), lambda i, bg: (bg[i], 0, 0)),
            pl.BlockSpec((1, _HIDDEN, _DIM), lambda i, bg: (bg[i], 0, 0)),
            pl.BlockSpec((1, _DIM, _HIDDEN), lambda i, bg: (bg[i], 0, 0)),
        ],
        out_specs=pl.BlockSpec((_TB, _DIM), lambda i, bg: (i, 0)),
    )
    return pl.pallas_call(
        _moe_body,
        grid_spec=grid_spec,
        out_shape=jax.ShapeDtypeStruct((_CAP, _DIM), jnp.float32),
        compiler_params=pltpu.CompilerParams(
            dimension_semantics=("parallel",),
            vmem_limit_bytes=128 * 1024 * 1024),
    )(bg, xs, w1, w2, w3)


_NC = 2
_NS = 16
_NW = _NC * _NS
_PER_W = _TOKENS // _NW
_CHUNK = 128


def _vector_mesh():
    return plsc.VectorSubcoreMesh(
        core_axis_name="core", subcore_axis_name="subcore")


def _sc_scatter(x, pos1d):
    @pl.kernel(out_type=jax.ShapeDtypeStruct((_CAP, _DIM), jnp.float32),
               mesh=_vector_mesh(),
               scratch_types=[
                   pltpu.VMEM((_CHUNK,), jnp.int32),
                   pltpu.VMEM((_CHUNK, _DIM), jnp.float32),
                   pltpu.SemaphoreType.DMA,
               ])
    def kern(x_hbm, i_hbm, xs_hbm, idx_v, rows_v, sem):
        wid = (jax.lax.axis_index("core") * _NS
               + jax.lax.axis_index("subcore"))
        base = wid * _PER_W

        @pl.loop(0, _PER_W // _CHUNK)
        def _(j):
            b = base + j * _CHUNK
            pltpu.sync_copy(i_hbm.at[pl.ds(b, _CHUNK)], idx_v)
            pltpu.sync_copy(x_hbm.at[pl.ds(b, _CHUNK)], rows_v)
            pltpu.async_copy(rows_v, xs_hbm.at[idx_v], sem).wait()

    return kern(x, pos1d)


def _sc_gather(ys, pos1d):
    @pl.kernel(out_type=jax.ShapeDtypeStruct((_TOKENS, _DIM), jnp.float32),
               mesh=_vector_mesh(),
               scratch_types=[
                   pltpu.VMEM((_CHUNK,), jnp.int32),
                   pltpu.VMEM((_CHUNK, _DIM), jnp.float32),
                   pltpu.SemaphoreType.DMA,
               ])
    def kern(y_hbm, i_hbm, o_hbm, idx_v, rows_v, sem):
        wid = (jax.lax.axis_index("core") * _NS
               + jax.lax.axis_index("subcore"))
        base = wid * _PER_W

        @pl.loop(0, _PER_W // _CHUNK)
        def _(j):
            b = base + j * _CHUNK
            pltpu.sync_copy(i_hbm.at[pl.ds(b, _CHUNK)], idx_v)
            pltpu.async_copy(y_hbm.at[idx_v], rows_v, sem).wait()
            pltpu.sync_copy(rows_v, o_hbm.at[pl.ds(b, _CHUNK)])

    return kern(ys, pos1d)


def kernel(x, router_w, w1, w2, w3):
    pos, meta, aux = _route(x, router_w)
    pos1d = pos.reshape(_TOKENS)
    bg = meta[0, :_CAP_BLOCKS]
    xs = _sc_scatter(x, pos1d)
    ys = _moe(bg, xs, w1, w2, w3)
    out = _sc_gather(ys, pos1d)
    return out, aux[0, 0]
